# Initial kernel scaffold; baseline (speedup 1.0000x reference)
import jax
import jax.numpy as jnp
from jax import lax
from jax.experimental import pallas as pl
from jax.experimental.pallas import tpu as pltpu
from jax.experimental.pallas import tpu_sc as plsc

_BATCH = 16384
_FIELDS = 26
_DIM = 32
_B = _BATCH * _FIELDS

_NC = 2
_NS = 16
_NW = _NC * _NS

_GATHER = 128
_CHUNK = 1024
_G_PER_CHUNK = _CHUNK // _GATHER
_B_PER_W = _B // _NW
_N_CHUNKS = _B_PER_W // _CHUNK


def _gather_body(idx_hbm, table_hbm, out_hbm, idx_v, rows_v, sem):
    wid = lax.axis_index("s") * _NC + lax.axis_index("c")
    row_base = wid * (_B_PER_W // _GATHER)

    def chunk_step(c, carry):
        grp = row_base + c * _G_PER_CHUNK
        pltpu.sync_copy(idx_hbm.at[pl.ds(grp, _G_PER_CHUNK)], idx_v)
        for j in range(_G_PER_CHUNK):
            pltpu.async_copy(
                table_hbm.at[idx_v.at[j]],
                rows_v.at[pl.ds(j * _GATHER, _GATHER)],
                sem,
            )
        for j in range(_G_PER_CHUNK):
            pltpu.make_async_copy(
                table_hbm.at[idx_v.at[j]],
                rows_v.at[pl.ds(j * _GATHER, _GATHER)],
                sem,
            ).wait()
        pltpu.sync_copy(rows_v, out_hbm.at[pl.ds(grp * _GATHER, _CHUNK)])
        return carry

    lax.fori_loop(0, _N_CHUNKS, chunk_step, 0)


@jax.jit
def kernel(x, weight):
    idx2d = x.reshape(_B // _GATHER, _GATHER).astype(jnp.int32)
    mesh = plsc.VectorSubcoreMesh(
        core_axis_name="c", subcore_axis_name="s",
        num_cores=_NC, num_subcores=_NS,
    )
    out_flat = pl.kernel(
        _gather_body,
        out_type=jax.ShapeDtypeStruct((_B, _DIM), jnp.float32),
        mesh=mesh,
        scratch_types=[
            pltpu.VMEM((_G_PER_CHUNK, _GATHER), jnp.int32),
            pltpu.VMEM((_CHUNK, _DIM), jnp.float32),
            pltpu.SemaphoreType.DMA,
        ],
    )(idx2d, weight)
    return out_flat.reshape(_BATCH, _FIELDS, _DIM)


# speedup vs baseline: 1.5471x; 1.5471x over previous
import jax
import jax.numpy as jnp
from jax import lax
from jax.experimental import pallas as pl
from jax.experimental.pallas import tpu as pltpu
from jax.experimental.pallas import tpu_sc as plsc

_BATCH = 16384
_FIELDS = 26
_DIM = 32
_B = _BATCH * _FIELDS

_NC = 2
_NS = 16
_NW = _NC * _NS

_GATHER = 128
_CHUNK = 1024
_G_PER_CHUNK = _CHUNK // _GATHER
_B_PER_W = _B // _NW
_N_CHUNKS = _B_PER_W // _CHUNK


def _gather_body(idx_hbm, table_hbm, out_hbm, idx_v, rows_v, sem):
    wid = lax.axis_index("s") * _NC + lax.axis_index("c")
    row_base = wid * (_B_PER_W // _GATHER)

    def chunk_step(c, carry):
        grp = row_base + c * _G_PER_CHUNK
        pltpu.sync_copy(idx_hbm.at[pl.ds(grp, _G_PER_CHUNK)], idx_v)
        for j in range(_G_PER_CHUNK):
            pltpu.async_copy(
                table_hbm.at[idx_v.at[j]],
                rows_v.at[pl.ds(j * _GATHER, _GATHER)],
                sem,
            )
        for j in range(_G_PER_CHUNK):
            pltpu.make_async_copy(
                table_hbm.at[idx_v.at[j]],
                rows_v.at[pl.ds(j * _GATHER, _GATHER)],
                sem,
            ).wait()
        pltpu.sync_copy(rows_v, out_hbm.at[pl.ds(grp * _GATHER, _CHUNK)])
        return carry

    lax.fori_loop(0, _N_CHUNKS, chunk_step, 0)


@jax.jit
def kernel(x, weight):
    idx2d = x.reshape(_B // _GATHER, _GATHER).astype(jnp.int32)
    mesh = plsc.VectorSubcoreMesh(
        core_axis_name="c", subcore_axis_name="s",
        num_cores=_NC, num_subcores=_NS,
    )
    out_flat = pl.kernel(
        _gather_body,
        out_type=jax.ShapeDtypeStruct((_B, _DIM), jnp.float32),
        mesh=mesh,
        scratch_types=[
            pltpu.VMEM((_G_PER_CHUNK, _GATHER), jnp.int32),
            pltpu.VMEM((_CHUNK, _DIM), jnp.float32),
            pltpu.SemaphoreType.DMA,
        ],
        compiler_params=pltpu.CompilerParams(use_tc_tiling_on_sc=False),
    )(idx2d, weight)
    return out_flat.reshape(_BATCH, _FIELDS, _DIM)


# speedup vs baseline: 1.5757x; 1.0185x over previous
import jax
import jax.numpy as jnp
from jax import lax
from jax.experimental import pallas as pl
from jax.experimental.pallas import tpu as pltpu
from jax.experimental.pallas import tpu_sc as plsc

_BATCH = 16384
_FIELDS = 26
_DIM = 32
_B = _BATCH * _FIELDS

_NC = 2
_NS = 16
_NW = _NC * _NS

_GATHER = 128
_CHUNK = 1024
_G_PER_CHUNK = _CHUNK // _GATHER
_B_PER_W = _B // _NW
_N_CHUNKS = _B_PER_W // _CHUNK
_NBUF = 2


def _gather_body(idx_hbm, table_hbm, out_hbm, idx_v, rows_v, isem, gsem, osem):
    wid = lax.axis_index("s") * _NC + lax.axis_index("c")
    grp0 = wid * (_B_PER_W // _GATHER)

    def idx_cp(c):
        slot = c % _NBUF
        return pltpu.make_async_copy(
            idx_hbm.at[pl.ds(grp0 + c * _G_PER_CHUNK, _G_PER_CHUNK)],
            idx_v.at[slot], isem.at[slot])

    def out_cp(c):
        slot = c % _NBUF
        return pltpu.make_async_copy(
            rows_v.at[slot],
            out_hbm.at[pl.ds((grp0 + c * _G_PER_CHUNK) * _GATHER, _CHUNK)],
            osem.at[slot])

    def gather_cp(c, j):
        slot = c % _NBUF
        return pltpu.make_async_copy(
            table_hbm.at[idx_v.at[slot].at[j]],
            rows_v.at[slot].at[pl.ds(j * _GATHER, _GATHER)],
            gsem.at[slot])

    for p in range(_NBUF):
        idx_cp(p).start()

    for c in range(_N_CHUNKS + 1):
        if c < _N_CHUNKS:
            idx_cp(c).wait()
            if c >= _NBUF:
                out_cp(c - _NBUF).wait()
            for j in range(_G_PER_CHUNK):
                gather_cp(c, j).start()
        if c >= 1:
            for j in range(_G_PER_CHUNK):
                gather_cp(c - 1, j).wait()
            out_cp(c - 1).start()
            if c - 1 + _NBUF < _N_CHUNKS:
                idx_cp(c - 1 + _NBUF).start()

    for c in range(_N_CHUNKS - _NBUF, _N_CHUNKS):
        out_cp(c).wait()


@jax.jit
def kernel(x, weight):
    idx2d = x.reshape(_B // _GATHER, _GATHER).astype(jnp.int32)
    mesh = plsc.VectorSubcoreMesh(
        core_axis_name="c", subcore_axis_name="s",
        num_cores=_NC, num_subcores=_NS,
    )
    out_flat = pl.kernel(
        _gather_body,
        out_type=jax.ShapeDtypeStruct((_B, _DIM), jnp.float32),
        mesh=mesh,
        scratch_types=[
            pltpu.VMEM((_NBUF, _G_PER_CHUNK, _GATHER), jnp.int32),
            pltpu.VMEM((_NBUF, _CHUNK, _DIM), jnp.float32),
            pltpu.SemaphoreType.DMA((_NBUF,)),
            pltpu.SemaphoreType.DMA((_NBUF,)),
            pltpu.SemaphoreType.DMA((_NBUF,)),
        ],
        compiler_params=pltpu.CompilerParams(use_tc_tiling_on_sc=False),
    )(idx2d, weight)
    return out_flat.reshape(_BATCH, _FIELDS, _DIM)
